# Initial kernel scaffold; baseline (speedup 1.0000x reference)
import jax
import jax.numpy as jnp
from jax.experimental import pallas as pl


def kernel(s, t_s, t_e, edge_index_parent, edge_index_child, edge_index_relate, emb_table, W_p, a_src_p, a_dst_p, b_p, W_c, a_src_c, a_dst_c, b_c, W_l_r, W_r_r, b_r):
    raise NotImplementedError("write your pallas kernel here")


# speedup vs baseline: 87.7539x; 87.7539x over previous
import functools

import jax
import jax.numpy as jnp
from jax import lax
from jax.experimental import pallas as pl
from jax.experimental.pallas import tpu as pltpu
from jax.experimental.pallas import tpu_sc as plsc

N = 10000
D = 128
E = 320000
B = 1024

NC = 2
NS = 16
NW = NC * NS
EPT = E // NW
EV = EPT // 16
CAP = 2048
K = 32
PAD = 64
BP = B + PAD
ROWS_TOT = 3 * BP
ZCH = ROWS_TOT // K
SPT = B // NW
NEG = -3e38


def _tc1_body(emb_ref, wp, asp, adp, wc, asc, adc, out_ref):
    c0 = jnp.dot(wp[...], asp[...], preferred_element_type=jnp.float32)
    c1 = jnp.dot(wp[...], adp[...], preferred_element_type=jnp.float32)
    c2 = jnp.dot(wc[...], asc[...], preferred_element_type=jnp.float32)
    c3 = jnp.dot(wc[...], adc[...], preferred_element_type=jnp.float32)
    c4 = jnp.concatenate([c0, c1, c2, c3], axis=1)
    out_ref[...] = lax.dot_general(
        c4, emb_ref[...], (((0,), (1,)), ((), ())),
        preferred_element_type=jnp.float32)


_tc1 = pl.pallas_call(
    _tc1_body,
    out_shape=jax.ShapeDtypeStruct((4, N), jnp.float32),
)


def _sc_body(s_hbm, eip, eic, eir, emb_hbm, alph_hbm,
             acc_out, den_out, g_out, embs_out,
             acc_sh, pos_sh,
             alph_v, pos_v, src_v, dst_v, csrc, cpos, cex,
             denL, dred, rows_v, pidx, s_v, gbuf, sem):
    c = lax.axis_index("c")
    sid = lax.axis_index("s")
    wid = c * NS + sid
    lanes = lax.iota(jnp.int32, 16)
    z16 = jnp.zeros((16,), jnp.float32)

    pltpu.sync_copy(alph_hbm, alph_v)
    pltpu.sync_copy(s_hbm, s_v)

    for r in range(K):
        for q in range(D // 16):
            rows_v[r, pl.ds(q * 16, 16)] = z16
    for j in range(ZCH // NS + 1):
        ch = sid + j * NS

        @pl.when(ch < ZCH)
        def _():
            pltpu.sync_copy(rows_v, acc_sh.at[pl.ds(ch * K, K)])

    @pl.when(sid == 0)
    def _():
        def init(i, _):
            pos_v[pl.ds(i * 16, 16)] = jnp.full((16,), -1, jnp.int32)
            return 0

        lax.fori_loop(0, N // 16, init, 0)

        def scat(i, _):
            sv16 = s_v[pl.ds(i * 16, 16)]
            vals = jnp.full((16,), i * 16, jnp.int32) + lanes
            for l in range(16):
                plsc.store_scatter(pos_v, [sv16], vals, mask=lanes == l)
            return 0

        lax.fori_loop(0, B // 16, scat, 0)
        pltpu.sync_copy(pos_v, pos_sh)

    plsc.subcore_barrier()

    @pl.when(sid != 0)
    def _():
        pltpu.sync_copy(pos_sh, pos_v)

    so = wid * SPT
    pltpu.async_copy(emb_hbm.at[s_v.at[pl.ds(so, SPT)]], rows_v, sem).wait()
    pltpu.sync_copy(rows_v, embs_out.at[pl.ds(so, SPT)])
    for q in range(SPT // 16):
        gbuf[pl.ds(q * 16, 16)] = plsc.load_gather(
            pos_v, [s_v[pl.ds(so + q * 16, 16)]])
    pltpu.sync_copy(gbuf, g_out.at[pl.ds(so, SPT)])

    def col_max(t0):
        def bd(i, mv):
            return jnp.maximum(mv, alph_v[pl.ds(t0 * N + i * 16, 16)])

        mv = lax.fori_loop(0, N // 16, bd, jnp.full((16,), NEG, jnp.float32))
        return jnp.max(mv, axis=0)

    g_p = col_max(0) + col_max(1)
    g_c = col_max(2) + col_max(3)
    shifts = [g_p, g_c, jnp.float32(0.0)]

    def do_type(t, ei_hbm, is_gat):
        base = wid * EPT
        pltpu.sync_copy(ei_hbm.at[pl.ds(base, EPT)], src_v)
        pltpu.sync_copy(ei_hbm.at[pl.ds(E + base, EPT)], dst_v)

        def pre(i, _):
            sl = pl.ds(i * 16, 16)
            csrc[sl] = jnp.zeros((16,), jnp.int32)
            cpos[sl] = jnp.full((16,), B, jnp.int32)
            cex[sl] = z16
            return 0

        lax.fori_loop(0, CAP // 16, pre, 0)

        def zden(g, _):
            for r in range(16):
                denL[r, pl.ds(g * 16, 16)] = z16
            return 0

        lax.fori_loop(0, B // 16, zden, 0)

        G = shifts[t]

        def pass_a(i, cur):
            sl = pl.ds(i * 16, 16)
            src = src_v[sl]
            dst = dst_v[sl]
            pos = plsc.load_gather(pos_v, [dst])
            m = pos >= 0
            if is_gat:
                als = plsc.load_gather(alph_v, [src + (2 * t) * N])
                ald = plsc.load_gather(alph_v, [dst + (2 * t + 1) * N])
                e = als + ald
                e = jnp.where(e >= 0, e, 0.2 * e)
                ex = jnp.exp(e - G)
            else:
                ex = jnp.ones((16,), jnp.float32)
            plsc.addupdate_scatter(denL, [lanes, pos], ex, mask=m)
            cur = jnp.minimum(cur, CAP - 16)
            plsc.store_compressed(csrc.at[pl.ds(cur, 16)], src, mask=m)
            plsc.store_compressed(cpos.at[pl.ds(cur, 16)], pos, mask=m)
            if is_gat:
                plsc.store_compressed(cex.at[pl.ds(cur, 16)], ex, mask=m)
            return cur + jnp.sum(m.astype(jnp.int32), axis=0)

        cnt = lax.fori_loop(0, EV, pass_a, jnp.int32(0))
        nch = (cnt + (K - 1)) // K

        def pass_c(i, _):
            o = i * K
            for q in range(K // 16):
                pidx[0, pl.ds(q * 16, 16)] = (
                    cpos[pl.ds(o + q * 16, 16)] + t * BP)
            pltpu.async_copy(
                emb_hbm.at[csrc.at[pl.ds(o, K)]], rows_v, sem).wait()
            if is_gat:
                for jv in range(K // 16):
                    wv = cex[pl.ds(o + jv * 16, 16)]
                    for l in range(16):
                        w = wv[l]
                        for q in range(D // 16):
                            qs = pl.ds(q * 16, 16)
                            rows_v[jv * 16 + l, qs] = rows_v[jv * 16 + l, qs] * w
            pltpu.sync_copy(rows_v, acc_sh.at[pidx.at[0]], add=True)
            return 0

        lax.fori_loop(0, nch, pass_c, 0)

        def dr(g, _):
            sl = pl.ds(g * 16, 16)
            a16 = denL[0, sl]
            for r in range(1, 16):
                a16 = a16 + denL[r, sl]
            dred[t, sl] = a16
            return 0

        lax.fori_loop(0, B // 16, dr, 0)

    do_type(0, eip, True)
    do_type(1, eic, True)
    do_type(2, eir, False)

    pltpu.sync_copy(dred, den_out.at[c, sid])

    plsc.subcore_barrier()

    for t in range(3):
        pltpu.sync_copy(
            acc_sh.at[pl.ds(t * BP + sid * (B // NS), B // NS)],
            acc_out.at[c, t, pl.ds(sid * (B // NS), B // NS)])


_sc_kernel = functools.partial(
    pl.kernel,
    out_type=(
        jax.ShapeDtypeStruct((NC, 3, B, D), jnp.float32),
        jax.ShapeDtypeStruct((NC, NS, 3, B), jnp.float32),
        jax.ShapeDtypeStruct((B,), jnp.int32),
        jax.ShapeDtypeStruct((B, D), jnp.float32),
    ),
    mesh=plsc.VectorSubcoreMesh(
        core_axis_name="c", subcore_axis_name="s",
        num_cores=NC, num_subcores=NS),
    compiler_params=pltpu.CompilerParams(needs_layout_passes=False),
    scratch_types=[
        pltpu.VMEM_SHARED((ROWS_TOT, D), jnp.float32),
        pltpu.VMEM_SHARED((N,), jnp.int32),
        pltpu.VMEM((4 * N,), jnp.float32),
        pltpu.VMEM((N,), jnp.int32),
        pltpu.VMEM((EPT,), jnp.int32),
        pltpu.VMEM((EPT,), jnp.int32),
        pltpu.VMEM((CAP,), jnp.int32),
        pltpu.VMEM((CAP,), jnp.int32),
        pltpu.VMEM((CAP,), jnp.float32),
        pltpu.VMEM((16, B), jnp.float32),
        pltpu.VMEM((3, B), jnp.float32),
        pltpu.VMEM((K, D), jnp.float32),
        pltpu.VMEM((1, K), jnp.int32),
        pltpu.VMEM((B,), jnp.int32),
        pltpu.VMEM((SPT,), jnp.int32),
        pltpu.SemaphoreType.DMA,
    ],
)(_sc_body)


def _tc2_body(acc_ref, den_ref, gt_ref, embs_ref,
              wp, bp, wc, bc, wl, wr, br, out_ref):
    num = acc_ref[0] + acc_ref[1]
    den = jnp.sum(den_ref[...], axis=0)

    def tmm(w, x):
        return lax.dot_general(
            w, x, (((0,), (1,)), ((), ())),
            preferred_element_type=jnp.float32)

    o_p = tmm(wp[...], num[0:B]) / (den[0:1, :] + 1e-16) + bp[...]
    o_c = tmm(wc[...], num[B:2 * B]) / (den[1:2, :] + 1e-16) + bc[...]
    o_r = (tmm(wl[...], num[2 * B:3 * B]) / jnp.maximum(den[2:3, :], 1.0)
           + tmm(wr[...], embs_ref[...]) + br[...])
    comb = (o_p + o_c + o_r) * jnp.float32(1.0 / 3.0)
    oht = (lax.broadcasted_iota(jnp.int32, (B, B), 0)
           == gt_ref[...]).astype(jnp.float32)
    out_ref[...] = jnp.dot(comb, oht, preferred_element_type=jnp.float32)


_tc2 = pl.pallas_call(
    _tc2_body,
    out_shape=jax.ShapeDtypeStruct((D, B), jnp.float32),
)


def kernel(s, t_s, t_e, edge_index_parent, edge_index_child,
           edge_index_relate, emb_table, W_p, a_src_p, a_dst_p, b_p,
           W_c, a_src_c, a_dst_c, b_c, W_l_r, W_r_r, b_r):
    s = s.astype(jnp.int32)
    eip = edge_index_parent.astype(jnp.int32)
    eic = edge_index_child.astype(jnp.int32)
    eir = edge_index_relate.astype(jnp.int32)

    alph = _tc1(emb_table, W_p, a_src_p.reshape(D, 1), a_dst_p.reshape(D, 1),
                W_c, a_src_c.reshape(D, 1), a_dst_c.reshape(D, 1))
    acc, den, g, embs = _sc_kernel(
        s, eip.reshape(2 * E), eic.reshape(2 * E), eir.reshape(2 * E),
        emb_table, alph.reshape(4 * N))
    out_t = _tc2(acc.reshape(NC, 3 * B, D), den.reshape(NC * NS, 3, B),
                 g.reshape(1, B), embs,
                 W_p, b_p.reshape(D, 1), W_c, b_c.reshape(D, 1),
                 W_l_r, W_r_r, b_r.reshape(D, 1))
    return out_t.T[None]
